# baseline (speedup 1.0000x reference)
import functools

import jax
import jax.numpy as jnp
from jax import lax
from jax.experimental import pallas as pl
from jax.experimental.pallas import tpu as pltpu
from jax.experimental.pallas import tpu_sc as plsc

_THETA = 0.0625
_TOPK = 8
_N = 4096
_LQ = 8
_BSZ = 64
_ROWS = _LQ * _BSZ
_NW = 32
_RPW = _ROWS // _NW
_NCH = _N // 16
_BIG = 1 << 30


_BPG = 4


def _tc_body(q_ref, k_ref, att_ref, *, b0):
    bstep = pl.program_id(0)
    for u in range(_BPG):
        q = q_ref[b0 + bstep * _BPG + u]
        k = k_ref[u]
        s = lax.dot_general(
            q, k, (((1,), (1,)), ((), ())), preferred_element_type=jnp.float32
        ) * _THETA
        m = jnp.max(s, axis=-1, keepdims=True)
        e = jnp.exp(s - m)
        att_ref[u] = e / jnp.sum(e, axis=-1, keepdims=True)


def _tc_attention(qT, keys, b0, nb):
    bsz, n, dk = keys.shape
    lq = qT.shape[1]
    return pl.pallas_call(
        functools.partial(_tc_body, b0=b0),
        grid=(nb // _BPG,),
        in_specs=[
            pl.BlockSpec((bsz, lq, dk), lambda b: (0, 0, 0)),
            pl.BlockSpec((_BPG, n, dk), lambda b, b0=b0: (b + b0 // _BPG, 0, 0)),
        ],
        out_specs=pl.BlockSpec((_BPG, lq, n), lambda b: (b, 0, 0)),
        out_shape=jax.ShapeDtypeStruct((nb, lq, n), jnp.float32),
    )(qT, keys)


def _sc_topk_body(att_hbm, idx_hbm, rows_v, cand_v, out_v, *, rpw):
    iota = lax.broadcasted_iota(jnp.int32, (16,), 0)
    wid = lax.axis_index("s") * 2 + lax.axis_index("c")
    base = wid * (rpw * _N)
    pltpu.sync_copy(att_hbm.at[pl.ds(base, rpw * _N)], rows_v)

    def row_body(r, _):
        r0 = r * _N

        def p1(i, acc):
            b = r0 + i * 128
            for t in range(8):
                acc = jnp.maximum(acc, rows_v[pl.ds(b + t * 16, 16)])
            return acc

        lanemax = lax.fori_loop(0, _NCH // 8, p1, jnp.full((16,), -1.0, jnp.float32))

        sv = plsc.sort_key_val(lanemax, iota)
        if isinstance(sv, (tuple, list)):
            sv = sv[0]
        thr = jnp.max(jnp.where(iota == 8, sv, -1.0))

        def p2(i, cnts):
            b = r0 + i * 128
            for t in range(8):
                c = rows_v[pl.ds(b + t * 16, 16)]
                msk = c >= thr
                plsc.store_scatter(
                    cand_v, [iota * _NCH + cnts], iota + i * 128 + t * 16, mask=msk
                )
                cnts = cnts + msk.astype(jnp.int32)
            return cnts

        cnts = lax.fori_loop(0, _NCH // 8, p2, jnp.zeros((16,), jnp.int32))
        maxcnt = jnp.max(cnts)

        picked = []
        for _j in range(_TOPK):
            def p3(p, st, picked=tuple(picked)):
                bv, bi = st
                valid = p < cnts
                iv = plsc.load_gather(cand_v, [iota * _NCH + p], mask=valid)
                v = plsc.load_gather(rows_v, [r0 + iv], mask=valid)
                v = jnp.where(valid, v, -1.0)
                for q in picked:
                    v = jnp.where(iv == q, -1.0, v)
                upd = v > bv
                return jnp.where(upd, v, bv), jnp.where(upd, iv, bi)

            bv, bi = lax.fori_loop(
                0, maxcnt, p3,
                (jnp.full((16,), -0.5, jnp.float32), jnp.zeros((16,), jnp.int32)),
            )
            g = jnp.max(bv)
            picked.append(jnp.min(jnp.where(bv == g, bi, _BIG)))

        pv = jnp.zeros((16,), jnp.int32)
        for j, q in enumerate(picked):
            pv = jnp.where(iota == j, q, pv)
        plsc.store_compressed(out_v.at[pl.ds(r * _TOPK, 16)], pv, mask=iota < _TOPK)
        return 0

    lax.fori_loop(0, rpw, row_body, 0)
    pltpu.sync_copy(
        out_v.at[pl.ds(0, rpw * _TOPK)],
        idx_hbm.at[pl.ds(wid * rpw * _TOPK, rpw * _TOPK)],
    )


def _make_sc_topk(rows):
    rpw = rows // _NW
    return functools.partial(
        pl.kernel,
        mesh=plsc.VectorSubcoreMesh(core_axis_name="c", subcore_axis_name="s"),
        compiler_params=pltpu.CompilerParams(needs_layout_passes=False),
        out_type=jax.ShapeDtypeStruct((rows * _TOPK,), jnp.int32),
        scratch_types=[
            pltpu.VMEM((rpw * _N,), jnp.float32),
            pltpu.VMEM((_N,), jnp.int32),
            pltpu.VMEM((rpw * _TOPK + 8,), jnp.int32),
        ],
    )(functools.partial(_sc_topk_body, rpw=rpw))


_NSLICE = 4
_SB = _BSZ // _NSLICE
_sc_topk_slice = _make_sc_topk(_SB * _LQ)


@jax.jit
def kernel(query, keys):
    qT = jnp.transpose(query, (2, 0, 1))
    atts, idxs = [], []
    for i in range(_NSLICE):
        att_i = _tc_attention(qT, keys, i * _SB, _SB)
        idxs.append(_sc_topk_slice(att_i.reshape(-1)))
        atts.append(att_i)
    att = jnp.concatenate(atts, axis=0)
    idx = jnp.concatenate(idxs).reshape(_BSZ, _LQ, _TOPK)
    return jnp.transpose(att, (1, 0, 2)), jnp.transpose(idx, (2, 1, 0))


# speedup vs baseline: 1.0038x; 1.0038x over previous
import functools

import jax
import jax.numpy as jnp
from jax import lax
from jax.experimental import pallas as pl
from jax.experimental.pallas import tpu as pltpu
from jax.experimental.pallas import tpu_sc as plsc

_THETA = 0.0625
_TOPK = 8
_N = 4096
_LQ = 8
_BSZ = 64
_ROWS = _LQ * _BSZ
_NW = 32
_RPW = _ROWS // _NW
_NCH = _N // 16
_BIG = 1 << 30


_BPG = 2


def _tc_body(q_ref, k_ref, att_ref, *, b0):
    bstep = pl.program_id(0)
    for u in range(_BPG):
        q = q_ref[b0 + bstep * _BPG + u]
        k = k_ref[u]
        s = lax.dot_general(
            q, k, (((1,), (1,)), ((), ())), preferred_element_type=jnp.float32
        ) * _THETA
        m = jnp.max(s, axis=-1, keepdims=True)
        e = jnp.exp(s - m)
        att_ref[u] = e / jnp.sum(e, axis=-1, keepdims=True)


def _tc_attention(qT, keys, b0, nb):
    bsz, n, dk = keys.shape
    lq = qT.shape[1]
    return pl.pallas_call(
        functools.partial(_tc_body, b0=b0),
        grid=(nb // _BPG,),
        in_specs=[
            pl.BlockSpec((bsz, lq, dk), lambda b: (0, 0, 0)),
            pl.BlockSpec((_BPG, n, dk), lambda b, b0=b0: (b + b0 // _BPG, 0, 0)),
        ],
        out_specs=pl.BlockSpec((_BPG, lq, n), lambda b: (b, 0, 0)),
        out_shape=jax.ShapeDtypeStruct((nb, lq, n), jnp.float32),
    )(qT, keys)


def _sc_topk_body(att_hbm, idx_hbm, rows_v, cand_v, out_v, *, rpw):
    iota = lax.broadcasted_iota(jnp.int32, (16,), 0)
    wid = lax.axis_index("s") * 2 + lax.axis_index("c")
    base = wid * (rpw * _N)
    pltpu.sync_copy(att_hbm.at[pl.ds(base, rpw * _N)], rows_v)

    def row_body(r, _):
        r0 = r * _N

        def p1(i, acc):
            b = r0 + i * 128
            for t in range(8):
                acc = jnp.maximum(acc, rows_v[pl.ds(b + t * 16, 16)])
            return acc

        lanemax = lax.fori_loop(0, _NCH // 8, p1, jnp.full((16,), -1.0, jnp.float32))

        sv = plsc.sort_key_val(lanemax, iota)
        if isinstance(sv, (tuple, list)):
            sv = sv[0]
        thr = jnp.max(jnp.where(iota == 8, sv, -1.0))

        def p2(i, cnts):
            b = r0 + i * 128
            for t in range(8):
                c = rows_v[pl.ds(b + t * 16, 16)]
                msk = c >= thr
                plsc.store_scatter(
                    cand_v, [iota * _NCH + cnts], iota + i * 128 + t * 16, mask=msk
                )
                cnts = cnts + msk.astype(jnp.int32)
            return cnts

        cnts = lax.fori_loop(0, _NCH // 8, p2, jnp.zeros((16,), jnp.int32))
        maxcnt = jnp.max(cnts)

        picked = []
        for _j in range(_TOPK):
            def p3(p, st, picked=tuple(picked)):
                bv, bi = st
                valid = p < cnts
                iv = plsc.load_gather(cand_v, [iota * _NCH + p], mask=valid)
                v = plsc.load_gather(rows_v, [r0 + iv], mask=valid)
                v = jnp.where(valid, v, -1.0)
                for q in picked:
                    v = jnp.where(iv == q, -1.0, v)
                upd = v > bv
                return jnp.where(upd, v, bv), jnp.where(upd, iv, bi)

            bv, bi = lax.fori_loop(
                0, maxcnt, p3,
                (jnp.full((16,), -0.5, jnp.float32), jnp.zeros((16,), jnp.int32)),
            )
            g = jnp.max(bv)
            picked.append(jnp.min(jnp.where(bv == g, bi, _BIG)))

        pv = jnp.zeros((16,), jnp.int32)
        for j, q in enumerate(picked):
            pv = jnp.where(iota == j, q, pv)
        plsc.store_compressed(out_v.at[pl.ds(r * _TOPK, 16)], pv, mask=iota < _TOPK)
        return 0

    lax.fori_loop(0, rpw, row_body, 0)
    pltpu.sync_copy(
        out_v.at[pl.ds(0, rpw * _TOPK)],
        idx_hbm.at[pl.ds(wid * rpw * _TOPK, rpw * _TOPK)],
    )


def _make_sc_topk(rows):
    rpw = rows // _NW
    return functools.partial(
        pl.kernel,
        mesh=plsc.VectorSubcoreMesh(core_axis_name="c", subcore_axis_name="s"),
        compiler_params=pltpu.CompilerParams(needs_layout_passes=False),
        out_type=jax.ShapeDtypeStruct((rows * _TOPK,), jnp.int32),
        scratch_types=[
            pltpu.VMEM((rpw * _N,), jnp.float32),
            pltpu.VMEM((_N,), jnp.int32),
            pltpu.VMEM((rpw * _TOPK + 8,), jnp.int32),
        ],
    )(functools.partial(_sc_topk_body, rpw=rpw))


_SLICES = (24, 16, 16, 8)
_sc_topk_fns = {sb: _make_sc_topk(sb * _LQ) for sb in set(_SLICES)}


@jax.jit
def kernel(query, keys):
    qT = jnp.transpose(query, (2, 0, 1))
    atts, idxs = [], []
    b0 = 0
    for sb in _SLICES:
        att_i = _tc_attention(qT, keys, b0, sb)
        idxs.append(_sc_topk_fns[sb](att_i.reshape(-1)))
        atts.append(att_i)
        b0 += sb
    att = jnp.concatenate(atts, axis=0)
    idx = jnp.concatenate(idxs).reshape(_BSZ, _LQ, _TOPK)
    return jnp.transpose(att, (1, 0, 2)), jnp.transpose(idx, (2, 1, 0))


# speedup vs baseline: 1.0349x; 1.0310x over previous
import functools

import jax
import jax.numpy as jnp
from jax import lax
from jax.experimental import pallas as pl
from jax.experimental.pallas import tpu as pltpu
from jax.experimental.pallas import tpu_sc as plsc

_THETA = 0.0625
_TOPK = 8
_N = 4096
_LQ = 8
_BSZ = 64
_ROWS = _LQ * _BSZ
_NW = 32
_RPW = _ROWS // _NW
_NCH = _N // 16
_BIG = 1 << 30


_BPG = 2


def _tc_body(q_ref, k_ref, att_ref, *, b0):
    bstep = pl.program_id(0)
    for u in range(_BPG):
        q = q_ref[b0 + bstep * _BPG + u]
        k = k_ref[u]
        s = lax.dot_general(
            q, k, (((1,), (1,)), ((), ())), preferred_element_type=jnp.float32
        ) * _THETA
        m = jnp.max(s, axis=-1, keepdims=True)
        e = jnp.exp(s - m)
        att_ref[u] = e / jnp.sum(e, axis=-1, keepdims=True)


def _tc_attention(qT, keys, b0, nb):
    bsz, n, dk = keys.shape
    lq = qT.shape[1]
    return pl.pallas_call(
        functools.partial(_tc_body, b0=b0),
        grid=(nb // _BPG,),
        in_specs=[
            pl.BlockSpec((bsz, lq, dk), lambda b: (0, 0, 0)),
            pl.BlockSpec((_BPG, n, dk), lambda b, b0=b0: (b + b0 // _BPG, 0, 0)),
        ],
        out_specs=pl.BlockSpec((_BPG, lq, n), lambda b: (b, 0, 0)),
        out_shape=jax.ShapeDtypeStruct((nb, lq, n), jnp.float32),
    )(qT, keys)


def _sc_topk_body(att_hbm, idx_hbm, rows_v, cand_v, out_v, *, rpw):
    iota = lax.broadcasted_iota(jnp.int32, (16,), 0)
    wid = lax.axis_index("s") * 2 + lax.axis_index("c")
    base = wid * (rpw * _N)
    pltpu.sync_copy(att_hbm.at[pl.ds(base, rpw * _N)], rows_v)

    def row_body(r, _):
        r0 = r * _N

        def p1(i, acc):
            b = r0 + i * 128
            for t in range(8):
                acc = jnp.maximum(acc, rows_v[pl.ds(b + t * 16, 16)])
            return acc

        lanemax = lax.fori_loop(0, _NCH // 8, p1, jnp.full((16,), -1.0, jnp.float32))

        sv = plsc.sort_key_val(lanemax, iota)
        if isinstance(sv, (tuple, list)):
            sv = sv[0]
        thr = jnp.max(jnp.where(iota == 8, sv, -1.0))

        def p2(i, cnts):
            b = r0 + i * 128
            for t in range(8):
                c = rows_v[pl.ds(b + t * 16, 16)]
                msk = c >= thr
                plsc.store_scatter(
                    cand_v, [iota * _NCH + cnts], iota + i * 128 + t * 16, mask=msk
                )
                cnts = cnts + msk.astype(jnp.int32)
            return cnts

        cnts = lax.fori_loop(0, _NCH // 8, p2, jnp.zeros((16,), jnp.int32))
        maxcnt = jnp.max(cnts)

        picked = []
        for _j in range(_TOPK):
            def p3(p, st, picked=tuple(picked)):
                bv, bi = st
                valid = p < cnts
                iv = plsc.load_gather(cand_v, [iota * _NCH + p], mask=valid)
                v = plsc.load_gather(rows_v, [r0 + iv], mask=valid)
                v = jnp.where(valid, v, -1.0)
                for q in picked:
                    v = jnp.where(iv == q, -1.0, v)
                upd = v > bv
                return jnp.where(upd, v, bv), jnp.where(upd, iv, bi)

            bv, bi = lax.fori_loop(
                0, maxcnt, p3,
                (jnp.full((16,), -0.5, jnp.float32), jnp.zeros((16,), jnp.int32)),
            )
            g = jnp.max(bv)
            picked.append(jnp.min(jnp.where(bv == g, bi, _BIG)))

        pv = jnp.zeros((16,), jnp.int32)
        for j, q in enumerate(picked):
            pv = jnp.where(iota == j, q, pv)
        plsc.store_compressed(out_v.at[pl.ds(r * _TOPK, 16)], pv, mask=iota < _TOPK)
        return 0

    lax.fori_loop(0, rpw, row_body, 0)
    pltpu.sync_copy(
        out_v.at[pl.ds(0, rpw * _TOPK)],
        idx_hbm.at[pl.ds(wid * rpw * _TOPK, rpw * _TOPK)],
    )


def _make_sc_topk(rows):
    rpw = rows // _NW
    return functools.partial(
        pl.kernel,
        mesh=plsc.VectorSubcoreMesh(core_axis_name="c", subcore_axis_name="s"),
        compiler_params=pltpu.CompilerParams(needs_layout_passes=False),
        out_type=jax.ShapeDtypeStruct((rows * _TOPK,), jnp.int32),
        scratch_types=[
            pltpu.VMEM((rpw * _N,), jnp.float32),
            pltpu.VMEM((_N,), jnp.int32),
            pltpu.VMEM((rpw * _TOPK + 8,), jnp.int32),
        ],
    )(functools.partial(_sc_topk_body, rpw=rpw))


_SLICES = (16, 16, 16, 16)
_sc_topk_fns = {sb: _make_sc_topk(sb * _LQ) for sb in set(_SLICES)}


@jax.jit
def kernel(query, keys):
    qT = jnp.transpose(query, (2, 0, 1))
    atts, idxs = [], []
    b0 = 0
    for sb in _SLICES:
        att_i = _tc_attention(qT, keys, b0, sb)
        idxs.append(_sc_topk_fns[sb](att_i.reshape(-1)))
        atts.append(att_i)
        b0 += sb
    att = jnp.concatenate(atts, axis=0)
    idx = jnp.concatenate(idxs).reshape(_BSZ, _LQ, _TOPK)
    return jnp.transpose(att, (1, 0, 2)), jnp.transpose(idx, (2, 1, 0))


# speedup vs baseline: 1.0363x; 1.0013x over previous
import functools

import jax
import jax.numpy as jnp
from jax import lax
from jax.experimental import pallas as pl
from jax.experimental.pallas import tpu as pltpu
from jax.experimental.pallas import tpu_sc as plsc

_THETA = 0.0625
_TOPK = 8
_N = 4096
_LQ = 8
_BSZ = 64
_ROWS = _LQ * _BSZ
_NW = 32
_RPW = _ROWS // _NW
_NCH = _N // 16
_BIG = 1 << 30


_BPG = 2


def _tc_body(q_ref, k_ref, att_ref, *, b0):
    bstep = pl.program_id(0)
    for u in range(_BPG):
        q = q_ref[b0 + bstep * _BPG + u]
        k = k_ref[u]
        s = lax.dot_general(
            q, k, (((1,), (1,)), ((), ())), preferred_element_type=jnp.float32
        ) * _THETA
        m = jnp.max(s, axis=-1, keepdims=True)
        e = jnp.exp(s - m)
        att_ref[u] = e / jnp.sum(e, axis=-1, keepdims=True)


def _tc_attention(qT, keys, b0, nb):
    bsz, n, dk = keys.shape
    lq = qT.shape[1]
    return pl.pallas_call(
        functools.partial(_tc_body, b0=b0),
        grid=(nb // _BPG,),
        in_specs=[
            pl.BlockSpec((bsz, lq, dk), lambda b: (0, 0, 0)),
            pl.BlockSpec((_BPG, n, dk), lambda b, b0=b0: (b + b0 // _BPG, 0, 0)),
        ],
        out_specs=pl.BlockSpec((_BPG, lq, n), lambda b: (b, 0, 0)),
        out_shape=jax.ShapeDtypeStruct((nb, lq, n), jnp.float32),
    )(qT, keys)


def _sc_topk_body(att_hbm, idx_hbm, rows_v, cand_v, out_v, *, rpw):
    iota = lax.broadcasted_iota(jnp.int32, (16,), 0)
    wid = lax.axis_index("s") * 2 + lax.axis_index("c")
    base = wid * (rpw * _N)
    pltpu.sync_copy(att_hbm.at[pl.ds(base, rpw * _N)], rows_v)

    def row_body(r, _):
        r0 = r * _N

        def p1(i, acc):
            b = r0 + i * 128
            for t in range(8):
                acc = jnp.maximum(acc, rows_v[pl.ds(b + t * 16, 16)])
            return acc

        lanemax = lax.fori_loop(0, _NCH // 8, p1, jnp.full((16,), -1.0, jnp.float32))

        sv = plsc.sort_key_val(lanemax, iota)
        if isinstance(sv, (tuple, list)):
            sv = sv[0]
        thr = jnp.max(jnp.where(iota == 8, sv, -1.0))

        def p2(i, cnts):
            b = r0 + i * 128
            for t in range(8):
                c = rows_v[pl.ds(b + t * 16, 16)]
                msk = c >= thr
                plsc.store_scatter(
                    cand_v, [iota * _NCH + cnts], iota + i * 128 + t * 16, mask=msk
                )
                cnts = cnts + msk.astype(jnp.int32)
            return cnts

        cnts = lax.fori_loop(0, _NCH // 8, p2, jnp.zeros((16,), jnp.int32))
        maxcnt = jnp.max(cnts)

        picked = []
        for _j in range(_TOPK):
            def p3(h, st, picked=tuple(picked)):
                bv, bi = st
                for u in range(2):
                    p = h * 2 + u
                    valid = p < cnts
                    iv = plsc.load_gather(cand_v, [iota * _NCH + p], mask=valid)
                    v = plsc.load_gather(rows_v, [r0 + iv], mask=valid)
                    v = jnp.where(valid, v, -1.0)
                    for q in picked:
                        v = jnp.where(iv == q, -1.0, v)
                    upd = v > bv
                    bv = jnp.where(upd, v, bv)
                    bi = jnp.where(upd, iv, bi)
                return bv, bi

            bv, bi = lax.fori_loop(
                0, (maxcnt + 1) // 2, p3,
                (jnp.full((16,), -0.5, jnp.float32), jnp.zeros((16,), jnp.int32)),
            )
            g = jnp.max(bv)
            picked.append(jnp.min(jnp.where(bv == g, bi, _BIG)))

        pv = jnp.zeros((16,), jnp.int32)
        for j, q in enumerate(picked):
            pv = jnp.where(iota == j, q, pv)
        plsc.store_compressed(out_v.at[pl.ds(r * _TOPK, 16)], pv, mask=iota < _TOPK)
        return 0

    lax.fori_loop(0, rpw, row_body, 0)
    pltpu.sync_copy(
        out_v.at[pl.ds(0, rpw * _TOPK)],
        idx_hbm.at[pl.ds(wid * rpw * _TOPK, rpw * _TOPK)],
    )


def _make_sc_topk(rows):
    rpw = rows // _NW
    return functools.partial(
        pl.kernel,
        mesh=plsc.VectorSubcoreMesh(core_axis_name="c", subcore_axis_name="s"),
        compiler_params=pltpu.CompilerParams(needs_layout_passes=False),
        out_type=jax.ShapeDtypeStruct((rows * _TOPK,), jnp.int32),
        scratch_types=[
            pltpu.VMEM((rpw * _N,), jnp.float32),
            pltpu.VMEM((_N,), jnp.int32),
            pltpu.VMEM((rpw * _TOPK + 8,), jnp.int32),
        ],
    )(functools.partial(_sc_topk_body, rpw=rpw))


_SLICES = (16, 16, 16, 16)
_sc_topk_fns = {sb: _make_sc_topk(sb * _LQ) for sb in set(_SLICES)}


@jax.jit
def kernel(query, keys):
    qT = jnp.transpose(query, (2, 0, 1))
    atts, idxs = [], []
    b0 = 0
    for sb in _SLICES:
        att_i = _tc_attention(qT, keys, b0, sb)
        idxs.append(_sc_topk_fns[sb](att_i.reshape(-1)))
        atts.append(att_i)
        b0 += sb
    att = jnp.concatenate(atts, axis=0)
    idx = jnp.concatenate(idxs).reshape(_BSZ, _LQ, _TOPK)
    return jnp.transpose(att, (1, 0, 2)), jnp.transpose(idx, (2, 1, 0))
